# baseline (speedup 1.0000x reference)
import functools

import jax
import jax.numpy as jnp
import numpy as np
from jax import lax
from jax.experimental import pallas as pl
from jax.experimental.pallas import tpu as pltpu
from jax.experimental.pallas import tpu_sc as plsc

N_PREFIX_HASH_BINS = 100000
MAX_SEQ_LEN = 200
PRIME = (1 << 31) - 1
BINS1 = N_PREFIX_HASH_BINS - 1

_rng = np.random.RandomState(42)
_A = _rng.randint(1, PRIME, size=(MAX_SEQ_LEN,)).astype(np.int64)
_B = int(_rng.randint(0, PRIME))

_PAD_LEN = 208
_A_PAD = np.zeros((_PAD_LEN,), np.int64)
_A_PAD[:MAX_SEQ_LEN] = _A
_A_LO = (_A_PAD & 0xFFFF).astype(np.int32)
_A_HI = (_A_PAD >> 16).astype(np.int32)

_NC, _NS = 2, 16
_NW = _NC * _NS
_NCHUNK = _PAD_LEN // 16


def _make_sc_kernel(batch, seqlen):
    rows_per = batch // _NW
    mesh = plsc.VectorSubcoreMesh(core_axis_name="c", subcore_axis_name="s")

    @functools.partial(
        pl.kernel,
        out_type=jax.ShapeDtypeStruct((batch, 2 * seqlen), jnp.int32),
        mesh=mesh,
        compiler_params=pltpu.CompilerParams(needs_layout_passes=False),
        scratch_types=[
            pltpu.VMEM((rows_per + 1, MAX_SEQ_LEN), jnp.int32),
            pltpu.VMEM((rows_per, 2 * seqlen), jnp.int32),
            pltpu.VMEM((2, _PAD_LEN), jnp.int32),
            pltpu.VMEM((_PAD_LEN,), jnp.int32),
            pltpu.VMEM((_PAD_LEN,), jnp.int32),
        ],
    )
    def body(seq_hbm, alo_hbm, ahi_hbm, out_hbm, seq_v, out_v, row_ids,
             alo_v, ahi_v):
        _U16 = jnp.uint32(0xFFFF)
        _U15 = jnp.uint32(0x7FFF)
        _UP = jnp.uint32(PRIME)
        _UB = jnp.uint32(_B)
        _INV_BINS1 = jnp.float32(1.0 / BINS1)
        _IBINS1 = jnp.int32(BINS1)
        wid = lax.axis_index("s") * _NC + lax.axis_index("c")
        pltpu.sync_copy(seq_hbm.at[pl.ds(wid * rows_per, rows_per), :],
                        seq_v.at[pl.ds(0, rows_per), :])
        pltpu.sync_copy(alo_hbm, alo_v)
        pltpu.sync_copy(ahi_hbm, ahi_v)
        pos0 = lax.iota(jnp.int32, 16)
        zeros16 = pos0 * 0
        x_ro, x_co = [], []
        for j in range(_NCHUNK):
            w = 16 * j + pos0
            x_ro.append(w // jnp.int32(seqlen))
            x_co.append(w % jnp.int32(seqlen))

        def do_row(r, ridx):
            rfull = jnp.full((16,), r, jnp.int32)
            n = zeros16
            c02 = jnp.uint32(0)
            c1 = jnp.uint32(0)
            ids = []
            for j in range(_NCHUNK):
                x_i = plsc.load_gather(seq_v, [rfull + x_ro[j], x_co[j]])
                x = plsc.bitcast(x_i, jnp.uint32)
                a0 = plsc.bitcast(alo_v[pl.ds(16 * j, 16)], jnp.uint32)
                a1 = plsc.bitcast(ahi_v[pl.ds(16 * j, 16)], jnp.uint32)
                x0 = x & _U16
                x1 = x >> jnp.uint32(16)
                m00 = a0 * x0
                m10 = a1 * x0
                m01 = a0 * x1
                m11 = a1 * x1
                e02 = (m00 & _U16) + ((m10 >> jnp.uint32(16)) + m11) * jnp.uint32(2)
                e1 = (m00 >> jnp.uint32(16)) + (m10 & _U16) + m01
                l02 = plsc.cumsum(e02) + c02
                l1 = plsc.cumsum(e1) + c1
                c02 = c02 + jnp.sum(e02, dtype=jnp.uint32)
                c1 = c1 + jnp.sum(e1, dtype=jnp.uint32)
                s = l02 + _UB
                r1 = (s & _UP) + (s >> jnp.uint32(31))
                s16v = ((l1 & _U15) << jnp.uint32(16)) + (l1 >> jnp.uint32(15))
                acc = r1 + s16v
                h = (acc & _UP) + (acc >> jnp.uint32(31))
                h = jnp.where(h >= _UP, h - _UP, h)
                hi = plsc.bitcast(h, jnp.int32)
                q = (hi.astype(jnp.float32) * _INV_BINS1).astype(jnp.int32)
                rr = hi - q * _IBINS1
                rr = jnp.where(rr < 0, rr + _IBINS1, rr)
                rr = jnp.where(rr >= _IBINS1, rr - _IBINS1, rr)
                idv = rr + 1
                nzb = x_i != 0
                if 16 * (j + 1) > seqlen:
                    nzb = nzb & (pos0 < jnp.int32(seqlen - 16 * j))
                n = n + plsc.all_reduce_population_count(nzb)
                row_ids[ridx, pl.ds(16 * j, 16)] = idv
                ids.append(idv)
            last_idx = jnp.clip(n - 1, 0, seqlen - 1)
            last_vec = plsc.load_gather(
                row_ids, [jnp.full((16,), ridx, jnp.int32), last_idx])
            for j in range(_NCHUNK):
                posj = pos0 + jnp.int32(16 * j)
                fixed = jnp.where(posj >= n, last_vec, ids[j])
                cols = posj * 2
                msk = None
                if 16 * (j + 1) > seqlen:
                    msk = posj < jnp.int32(seqlen)
                    cols = jnp.minimum(cols, jnp.int32(2 * seqlen - 2))
                plsc.store_scatter(out_v, [rfull, cols], fixed, mask=msk)
                plsc.store_scatter(out_v, [rfull, cols + 1], zeros16,
                                   mask=msk)

        def row_body(rr, carry):
            do_row(rr * 2, 0)
            do_row(rr * 2 + 1, 1)
            return carry

        lax.fori_loop(jnp.int32(0), jnp.int32(rows_per // 2), row_body,
                      jnp.int32(0))
        pltpu.sync_copy(out_v,
                        out_hbm.at[pl.ds(wid * rows_per, rows_per), :])

    return body


def kernel(sequences):
    batch, seqlen = sequences.shape
    pairs = _make_sc_kernel(batch, seqlen)(
        sequences.astype(jnp.int32), jnp.asarray(_A_LO), jnp.asarray(_A_HI))
    return jax.lax.bitcast_convert_type(
        pairs.reshape(batch, seqlen, 2), jnp.int64)


# speedup vs baseline: 1.0179x; 1.0179x over previous
import functools

import jax
import jax.numpy as jnp
import numpy as np
from jax import lax
from jax.experimental import pallas as pl
from jax.experimental.pallas import tpu as pltpu
from jax.experimental.pallas import tpu_sc as plsc

N_PREFIX_HASH_BINS = 100000
MAX_SEQ_LEN = 200
PRIME = (1 << 31) - 1
BINS1 = N_PREFIX_HASH_BINS - 1

_rng = np.random.RandomState(42)
_A = _rng.randint(1, PRIME, size=(MAX_SEQ_LEN,)).astype(np.int64)
_B = int(_rng.randint(0, PRIME))

_PAD_LEN = 208
_A_PAD = np.zeros((_PAD_LEN,), np.int64)
_A_PAD[:MAX_SEQ_LEN] = _A
_A_LO = (_A_PAD & 0xFFFF).astype(np.int32)
_A_HI = (_A_PAD >> 16).astype(np.int32)

_NC, _NS = 2, 16
_NW = _NC * _NS
_NCHUNK = _PAD_LEN // 16


def _make_sc_kernel(batch, seqlen):
    rows_per = batch // _NW
    blk = rows_per * _PAD_LEN
    mesh = plsc.VectorSubcoreMesh(core_axis_name="c", subcore_axis_name="s")

    @functools.partial(
        pl.kernel,
        out_type=jax.ShapeDtypeStruct((batch, 2 * seqlen), jnp.int32),
        mesh=mesh,
        compiler_params=pltpu.CompilerParams(needs_layout_passes=False),
        scratch_types=[
            pltpu.VMEM((blk,), jnp.int32),
            pltpu.VMEM((rows_per, 2 * seqlen), jnp.int32),
            pltpu.VMEM((_PAD_LEN,), jnp.int32),
            pltpu.VMEM((_PAD_LEN,), jnp.int32),
        ],
    )
    def body(seq_hbm, alo_hbm, ahi_hbm, out_hbm, seq_v, out_v, alo_v, ahi_v):
        _U16 = jnp.uint32(0xFFFF)
        _U15 = jnp.uint32(0x7FFF)
        _UP = jnp.uint32(PRIME)
        _INV_BINS1 = jnp.float32(1.0 / BINS1)
        _IBINS1 = jnp.int32(BINS1)
        wid = lax.axis_index("s") * _NC + lax.axis_index("c")
        pltpu.sync_copy(seq_hbm.at[pl.ds(wid * blk, blk)], seq_v)
        pltpu.sync_copy(alo_hbm, alo_v)
        pltpu.sync_copy(ahi_hbm, ahi_v)
        pos0 = lax.iota(jnp.int32, 16)
        zeros16 = pos0 * 0
        cols_c = [jnp.minimum((pos0 + 16 * j) * 2, jnp.int32(2 * seqlen - 2))
                  for j in range(_NCHUNK)]

        def row_body(r, carry):
            base = r * _PAD_LEN
            rfull = jnp.full((16,), r, jnp.int32)
            n = zeros16
            c02 = jnp.uint32(_B & 0xFFFF)
            c1 = jnp.uint32(_B >> 16)
            for j in range(_NCHUNK):
                x_i = seq_v[pl.ds(base + 16 * j, 16)]
                x = plsc.bitcast(x_i, jnp.uint32)
                a0 = plsc.bitcast(alo_v[pl.ds(16 * j, 16)], jnp.uint32)
                a1 = plsc.bitcast(ahi_v[pl.ds(16 * j, 16)], jnp.uint32)
                x0 = x & _U16
                x1 = x >> jnp.uint32(16)
                m00 = a0 * x0
                m10 = a1 * x0
                m01 = a0 * x1
                m11 = a1 * x1
                e02 = (m00 & _U16) + ((m10 >> jnp.uint32(16)) + m11) * jnp.uint32(2)
                e1 = (m00 >> jnp.uint32(16)) + (m10 & _U16) + m01
                l02 = plsc.cumsum(e02) + c02
                l1 = plsc.cumsum(e1) + c1
                c02 = c02 + jnp.sum(e02, dtype=jnp.uint32)
                c1 = c1 + jnp.sum(e1, dtype=jnp.uint32)
                s16v = ((l1 & _U15) << jnp.uint32(16)) + (l1 >> jnp.uint32(15))
                acc = l02 + s16v
                h = (acc & _UP) + (acc >> jnp.uint32(31))
                h = jnp.where(h >= _UP, h - _UP, h)
                hi = plsc.bitcast(h, jnp.int32)
                q = (hi.astype(jnp.float32) * _INV_BINS1).astype(jnp.int32)
                rr = hi - q * _IBINS1
                rr = jnp.where(rr < 0, rr + _IBINS1, rr)
                rr = jnp.where(rr >= _IBINS1, rr - _IBINS1, rr)
                idv = rr + 1
                nzb = x_i != 0
                msk = None
                if 16 * (j + 1) > seqlen:
                    lanes_ok = pos0 < jnp.int32(seqlen - 16 * j)
                    nzb = nzb & lanes_ok
                    msk = lanes_ok
                n = n + plsc.all_reduce_population_count(nzb)
                plsc.store_scatter(out_v, [rfull, cols_c[j]], idv, mask=msk)
                plsc.store_scatter(out_v, [rfull, cols_c[j] + 1], zeros16,
                                   mask=msk)
            last_idx = jnp.clip(n - 1, 0, seqlen - 1)
            last_vec = plsc.load_gather(out_v, [rfull, last_idx * 2])
            n_s = jnp.max(n)

            def tail_body(k, carry2):
                posk = pos0 + k * 16
                m = (posk >= n) & (posk < jnp.int32(seqlen))
                ck = jnp.minimum(posk * 2, jnp.int32(2 * seqlen - 2))
                plsc.store_scatter(out_v, [rfull, ck], last_vec, mask=m)
                return carry2

            lax.fori_loop(n_s // jnp.int32(16), jnp.int32(_NCHUNK),
                          tail_body, jnp.int32(0))
            return carry

        lax.fori_loop(jnp.int32(0), jnp.int32(rows_per), row_body,
                      jnp.int32(0))
        pltpu.sync_copy(out_v,
                        out_hbm.at[pl.ds(wid * rows_per, rows_per), :])

    return body


def kernel(sequences):
    batch, seqlen = sequences.shape
    x = sequences.astype(jnp.int32)
    xp = jnp.pad(x, ((0, 0), (0, _PAD_LEN - seqlen)))
    pairs = _make_sc_kernel(batch, seqlen)(
        xp.reshape(-1), jnp.asarray(_A_LO), jnp.asarray(_A_HI))
    return jax.lax.bitcast_convert_type(
        pairs.reshape(batch, seqlen, 2), jnp.int64)
